# Initial kernel scaffold; baseline (speedup 1.0000x reference)
import jax
import jax.numpy as jnp
from jax.experimental import pallas as pl


def kernel(tokens, W_pos):
    raise NotImplementedError("write your pallas kernel here")


# speedup vs baseline: 2.2925x; 2.2925x over previous
import jax
import jax.numpy as jnp
from jax.experimental import pallas as pl

_BS = 256


def _bcast_copy_kernel(w_ref, o_ref):
    o_ref[...] = jnp.broadcast_to(w_ref[...][None, :, :], o_ref.shape)


def kernel(tokens, W_pos):
    batch, seq_len = tokens.shape
    d_model = W_pos.shape[1]
    grid = seq_len // _BS
    return pl.pallas_call(
        _bcast_copy_kernel,
        grid=(grid,),
        in_specs=[pl.BlockSpec((_BS, d_model), lambda s: (s, 0))],
        out_specs=pl.BlockSpec((batch, _BS, d_model), lambda s: (0, s, 0)),
        out_shape=jax.ShapeDtypeStruct((batch, seq_len, d_model), W_pos.dtype),
    )(W_pos)


# speedup vs baseline: 2.3504x; 1.0253x over previous
import jax
import jax.numpy as jnp
from jax.experimental import pallas as pl

_BS = 512


def _bcast_copy_kernel(w_ref, o_ref):
    o_ref[...] = jnp.broadcast_to(w_ref[...][None, :, :], o_ref.shape)


def kernel(tokens, W_pos):
    batch, seq_len = tokens.shape
    d_model = W_pos.shape[1]
    grid = seq_len // _BS
    return pl.pallas_call(
        _bcast_copy_kernel,
        grid=(grid,),
        in_specs=[pl.BlockSpec((_BS, d_model), lambda s: (s, 0))],
        out_specs=pl.BlockSpec((batch, _BS, d_model), lambda s: (0, s, 0)),
        out_shape=jax.ShapeDtypeStruct((batch, seq_len, d_model), W_pos.dtype),
    )(W_pos)
